# Initial kernel scaffold; baseline (speedup 1.0000x reference)
import jax
import jax.numpy as jnp
from jax.experimental import pallas as pl


def kernel(x, node_attr, edge_index, edge_label, node_emb, net_W, net_b, dev_W, dev_b, pin_emb, sage0_lW, sage0_lb, sage0_rW, sage1_lW, sage1_lb, sage1_rW, mlp_W1, mlp_b1, mlp_W2, mlp_b2, mlp_W3, mlp_b3):
    raise NotImplementedError("write your pallas kernel here")


# speedup vs baseline: 3.9325x; 3.9325x over previous
import functools

import jax
import jax.numpy as jnp
from jax import lax
from jax.experimental import pallas as pl
from jax.experimental.pallas import tpu as pltpu
from jax.experimental.pallas import tpu_sc as plsc

N = 50000
E = 800000
H = 64
HH = 32
B = 4096
NCLS = 3

NC = 2
NS = 16
K = 128

NP = 50176
NSL = NP // NS
ET = NP
EP = NS * ET
NBLK = ET // K


def _make_seg_sum(do_cnt: bool):
    mesh = plsc.VectorSubcoreMesh(core_axis_name="c", subcore_axis_name="s")
    out_type = [jax.ShapeDtypeStruct((NC, NP, HH), jnp.float32)]
    if do_cnt:
        out_type.append(jax.ShapeDtypeStruct((NP,), jnp.float32))
    CH = NSL // 8
    scratch = [
        pltpu.VMEM_SHARED((NP, HH), jnp.float32),
        pltpu.VMEM((K,), jnp.int32),
        pltpu.VMEM((K,), jnp.int32),
        pltpu.VMEM((K, HH), jnp.float32),
        pltpu.VMEM((CH, HH), jnp.float32),
        pltpu.SemaphoreType.DMA,
    ]
    if do_cnt:
        scratch.append(pltpu.VMEM_SHARED((NP,), jnp.float32))
        scratch.append(pltpu.VMEM((K,), jnp.float32))
        scratch.append(pltpu.VMEM((NSL,), jnp.float32))

    def body(tab, srcr, dstr, zrow, *rest):
        if do_cnt:
            (zcnt, ones_h, msum_out, cnt_out,
             acc, idxs, idxd, rows, stg, sem, cnt_sh, ones_v, cstg) = rest
        else:
            msum_out, acc, idxs, idxd, rows, stg, sem = rest
        cid = lax.axis_index("c")
        sid = lax.axis_index("s")
        sl = pl.ds(sid * NSL, NSL)

        pltpu.sync_copy(zrow, stg)
        for j in range(NSL // CH):
            pltpu.sync_copy(stg, acc.at[pl.ds(sid * NSL + j * CH, CH)])
        if do_cnt:
            @pl.when(cid == 0)
            def _():
                pltpu.sync_copy(zcnt.at[sl], cstg)
                pltpu.sync_copy(cstg, cnt_sh.at[sl])
                pltpu.sync_copy(ones_h, ones_v)
        plsc.subcore_barrier()

        def step(i, carry):
            base = sid * ET + i * K
            pltpu.sync_copy(srcr.at[pl.ds(base, K)], idxs)
            pltpu.sync_copy(dstr.at[pl.ds(base, K)], idxd)
            pltpu.async_copy(tab.at[cid].at[idxs], rows, sem).wait()
            pltpu.sync_copy(rows, acc.at[idxd], add=True)
            if do_cnt:
                @pl.when(cid == 0)
                def _():
                    pltpu.sync_copy(ones_v, cnt_sh.at[idxd], add=True)
            return carry

        lax.fori_loop(0, NBLK, step, 0)
        plsc.subcore_barrier()

        for j in range(NSL // CH):
            csl = pl.ds(sid * NSL + j * CH, CH)
            pltpu.sync_copy(acc.at[csl], stg)
            pltpu.sync_copy(stg, msum_out.at[cid].at[csl])
        if do_cnt:
            @pl.when(cid == 0)
            def _():
                pltpu.sync_copy(cnt_sh.at[sl], cstg)
                pltpu.sync_copy(cstg, cnt_out.at[sl])

    return pl.kernel(body, out_type=tuple(out_type), mesh=mesh,
                     scratch_types=scratch,
                     compiler_params=pltpu.CompilerParams(
                         use_tc_tiling_on_sc=False))


_seg_sum_cnt = _make_seg_sum(True)
_seg_sum = _make_seg_sum(False)


_BN = 6272


def _embed_body(nt_ref, attr_ref, nemb_ref, netWT_ref, devWT_ref, pemb_ref,
                zA_ref, zB_ref):
    nt = nt_ref[...][:, 0]
    attr = attr_ref[...]
    net_e = jnp.dot(attr, netWT_ref[...], preferred_element_type=jnp.float32)
    dev_e = jnp.dot(attr, devWT_ref[...], preferred_element_type=jnp.float32)
    pin_idx = attr[:, 0].astype(jnp.int32)
    poh = (pin_idx[:, None] ==
           lax.broadcasted_iota(jnp.int32, (_BN, 17), 1)).astype(jnp.float32)
    pin_e = jnp.dot(poh, pemb_ref[...], preferred_element_type=jnp.float32)
    noh = (nt[:, None] ==
           lax.broadcasted_iota(jnp.int32, (_BN, 4), 1)).astype(jnp.float32)
    zA = jnp.dot(noh, nemb_ref[...], preferred_element_type=jnp.float32)
    zero = jnp.zeros_like(net_e)
    attr_sel = jnp.where((nt == 0)[:, None], net_e,
               jnp.where((nt == 1)[:, None], dev_e,
               jnp.where((nt == 2)[:, None], pin_e, zero)))
    zA_ref[...] = zA
    zB_ref[...] = attr_sel


def _embed(nt, attr, node_emb, netWT, devWT, pin_emb):
    return pl.pallas_call(
        _embed_body,
        grid=(NP // _BN,),
        in_specs=[
            pl.BlockSpec((_BN, 1), lambda i: (i, 0)),
            pl.BlockSpec((_BN, 17), lambda i: (i, 0)),
            pl.BlockSpec((4, HH), lambda i: (0, 0)),
            pl.BlockSpec((17, HH), lambda i: (0, 0)),
            pl.BlockSpec((17, HH), lambda i: (0, 0)),
            pl.BlockSpec((17, HH), lambda i: (0, 0)),
        ],
        out_specs=[
            pl.BlockSpec((_BN, HH), lambda i: (i, 0)),
            pl.BlockSpec((_BN, HH), lambda i: (i, 0)),
        ],
        out_shape=[
            jax.ShapeDtypeStruct((NP, HH), jnp.float32),
            jax.ShapeDtypeStruct((NP, HH), jnp.float32),
        ],
    )(nt, attr, node_emb, netWT, devWT, pin_emb)


def _combine0_body(mA_ref, mB_ref, zA_ref, zB_ref, cnt_ref, lWT_ref, lb_ref,
                   rWT_ref, oA_ref, oB_ref, inv_ref):
    cnt = cnt_ref[...][:, 0]
    inv = 1.0 / jnp.maximum(cnt, 1.0)
    agg = jnp.concatenate([mA_ref[...], mB_ref[...]], axis=1) * inv[:, None]
    z = jnp.concatenate([zA_ref[...], zB_ref[...]], axis=1)
    o = jnp.dot(agg, lWT_ref[...], preferred_element_type=jnp.float32) \
        + lb_ref[...] \
        + jnp.dot(z, rWT_ref[...], preferred_element_type=jnp.float32)
    o = jnp.maximum(o, 0.0)
    oA_ref[...] = o[:, :HH]
    oB_ref[...] = o[:, HH:]
    inv_ref[...] = inv[:, None]


def _combine0(mA, mB, zA, zB, cnt, lWT, lb, rWT):
    return pl.pallas_call(
        _combine0_body,
        grid=(NP // _BN,),
        in_specs=[
            pl.BlockSpec((_BN, HH), lambda i: (i, 0)),
            pl.BlockSpec((_BN, HH), lambda i: (i, 0)),
            pl.BlockSpec((_BN, HH), lambda i: (i, 0)),
            pl.BlockSpec((_BN, HH), lambda i: (i, 0)),
            pl.BlockSpec((_BN, 1), lambda i: (i, 0)),
            pl.BlockSpec((H, H), lambda i: (0, 0)),
            pl.BlockSpec((1, H), lambda i: (0, 0)),
            pl.BlockSpec((H, H), lambda i: (0, 0)),
        ],
        out_specs=[
            pl.BlockSpec((_BN, HH), lambda i: (i, 0)),
            pl.BlockSpec((_BN, HH), lambda i: (i, 0)),
            pl.BlockSpec((_BN, 1), lambda i: (i, 0)),
        ],
        out_shape=[
            jax.ShapeDtypeStruct((NP, HH), jnp.float32),
            jax.ShapeDtypeStruct((NP, HH), jnp.float32),
            jax.ShapeDtypeStruct((NP, 1), jnp.float32),
        ],
    )(mA, mB, zA, zB, cnt, lWT, lb, rWT)


_B2 = 2 * B


def _final_body(mA_ref, mB_ref, zA_ref, zB_ref, inv_ref, lWT_ref, lb_ref,
                rWT_ref, W1T_ref, b1_ref, W2T_ref, b2_ref, W3T_ref, b3_ref,
                logits_ref, ge_ref):
    inv = inv_ref[...]
    agg = jnp.concatenate([mA_ref[...], mB_ref[...]], axis=1) * inv
    z = jnp.concatenate([zA_ref[...], zB_ref[...]], axis=1)
    z2 = jnp.dot(agg, lWT_ref[...], preferred_element_type=jnp.float32) \
        + lb_ref[...] \
        + jnp.dot(z, rWT_ref[...], preferred_element_type=jnp.float32)
    z2 = jnp.maximum(z2, 0.0)
    ge = jnp.concatenate([z2[:B], z2[B:]], axis=1)
    h = jnp.maximum(jnp.dot(ge, W1T_ref[...],
                            preferred_element_type=jnp.float32)
                    + b1_ref[...], 0.0)
    h = jnp.maximum(jnp.dot(h, W2T_ref[...],
                            preferred_element_type=jnp.float32)
                    + b2_ref[...], 0.0)
    logits_ref[...] = jnp.dot(h, W3T_ref[...],
                              preferred_element_type=jnp.float32) + b3_ref[...]
    ge_ref[...] = ge


def _final(mA, mB, zA, zB, inv, lWT, lb, rWT, W1T, b1, W2T, b2, W3T, b3):
    full = lambda s: pl.BlockSpec(s, lambda: tuple(0 for _ in s))
    return pl.pallas_call(
        _final_body,
        in_specs=[
            full((_B2, HH)), full((_B2, HH)), full((_B2, HH)), full((_B2, HH)),
            full((_B2, 1)),
            full((H, H)), full((1, H)), full((H, H)),
            full((2 * H, H)), full((1, H)), full((H, H)), full((1, H)),
            full((H, NCLS)), full((1, NCLS)),
        ],
        out_specs=[full((B, NCLS)), full((B, 2 * H))],
        out_shape=[
            jax.ShapeDtypeStruct((B, NCLS), jnp.float32),
            jax.ShapeDtypeStruct((B, 2 * H), jnp.float32),
        ],
    )(mA, mB, zA, zB, inv, lWT, lb, rWT, W1T, b1, W2T, b2, W3T, b3)


def kernel(x, node_attr, edge_index, edge_label, node_emb, net_W, net_b,
           dev_W, dev_b, pin_emb, sage0_lW, sage0_lb, sage0_rW, sage1_lW,
           sage1_lb, sage1_rW, mlp_W1, mlp_b1, mlp_W2, mlp_b2, mlp_W3,
           mlp_b3):
    f32 = jnp.float32
    nt = jnp.pad(x.astype(jnp.int32), ((0, NP - N), (0, 0)))
    attr = jnp.pad(node_attr, ((0, NP - N), (0, 0)))
    src = jnp.concatenate([edge_index[0].astype(jnp.int32),
                           jnp.full((EP - E,), N, jnp.int32)])
    dst = jnp.concatenate([edge_index[1].astype(jnp.int32),
                           jnp.full((EP - E,), N, jnp.int32)])
    zrow = jnp.zeros((NSL // 8, HH), f32)
    zcnt = jnp.zeros((NP,), f32)
    ones_h = jnp.ones((K,), f32)

    zA, zB = _embed(nt, attr, node_emb, net_W.T, dev_W.T, pin_emb)

    tab0 = jnp.stack([zA, zB])
    (msum0, cnt) = _seg_sum_cnt(tab0, src, dst, zrow, zcnt, ones_h)

    z1A, z1B, inv = _combine0(msum0[0], msum0[1], zA, zB, cnt.reshape(NP, 1),
                              sage0_lW.T, sage0_lb.reshape(1, H), sage0_rW.T)

    tab1 = jnp.stack([z1A, z1B])
    (msum1,) = _seg_sum(tab1, src, dst, zrow)

    logits, ge = _final(
        msum1[0, :_B2], msum1[1, :_B2], z1A[:_B2], z1B[:_B2], inv[:_B2],
        sage1_lW.T, sage1_lb.reshape(1, H), sage1_rW.T,
        mlp_W1.T, mlp_b1.reshape(1, H), mlp_W2.T, mlp_b2.reshape(1, H),
        mlp_W3.T, mlp_b3.reshape(1, NCLS))
    return (logits, ge)
